# baseline (speedup 1.0000x reference)
import jax
import jax.numpy as jnp
from jax.experimental import pallas as pl
from jax.experimental.pallas import tpu as pltpu

_EPS = 1e-05


def _fused_kernel(x_ref, w_ref, b_ref, o_ref, s1_ref, s2_ref, cnt_ref, sc_ref):
    h = pl.program_id(1)
    nh = pl.num_programs(1)
    hb_steps = nh // 2
    x = x_ref[...]
    m = jnp.sum(jnp.abs(x), axis=1, keepdims=True) != 0

    @pl.when(h < hb_steps)
    def _():
        mf = m.astype(x.dtype)
        p1 = jnp.sum(x, axis=2, keepdims=True)
        p2 = jnp.sum(x * x, axis=2, keepdims=True)
        pc = jnp.sum(mf, axis=2, keepdims=True)

        @pl.when(h == 0)
        def _():
            s1_ref[...] = p1
            s2_ref[...] = p2
            cnt_ref[...] = pc

        @pl.when(h != 0)
        def _():
            s1_ref[...] += p1
            s2_ref[...] += p2
            cnt_ref[...] += pc

    @pl.when(h >= hb_steps)
    def _():
        w = w_ref[...]
        b = b_ref[...]

        @pl.when(h == hb_steps)
        def _():
            s1 = jnp.sum(s1_ref[...], axis=3, keepdims=True)
            s2 = jnp.sum(s2_ref[...], axis=3, keepdims=True)
            cnt = jnp.sum(cnt_ref[...], axis=3, keepdims=True)
            safe = jnp.maximum(cnt, 1.0)
            mean = s1 / safe
            var = jnp.maximum(s2 / safe - mean * mean, 0.0)
            rstd = jax.lax.rsqrt(var + _EPS)
            do_norm = cnt > 1.0
            ws = w * rstd
            sc_ref[0:1] = jnp.where(do_norm, ws, w)
            sc_ref[1:2] = jnp.where(do_norm, b - mean * ws, b)

        scale = jnp.where(m, sc_ref[0:1], w)
        shift = jnp.where(m, sc_ref[1:2], b)
        o_ref[...] = x * scale + shift


def kernel(x, weight, bias):
    B, C, H, W = x.shape
    hb = min(64, H)
    HB = H // hb
    w4 = weight.reshape(1, C, 1, 1)
    b4 = bias.reshape(1, C, 1, 1)

    def x_idx(b, h):
        return (b, 0, jnp.where(h < HB, h, 2 * HB - 1 - h), 0)

    def o_idx(b, h):
        return (b, 0, jnp.where(h < HB, HB - 1, 2 * HB - 1 - h), 0)

    out = pl.pallas_call(
        _fused_kernel,
        grid=(B, 2 * HB),
        in_specs=[
            pl.BlockSpec((1, C, hb, W), x_idx),
            pl.BlockSpec((1, C, 1, 1), lambda b, h: (0, 0, 0, 0)),
            pl.BlockSpec((1, C, 1, 1), lambda b, h: (0, 0, 0, 0)),
        ],
        out_specs=pl.BlockSpec((1, C, hb, W), o_idx),
        out_shape=jax.ShapeDtypeStruct((B, C, H, W), x.dtype),
        scratch_shapes=[
            pltpu.VMEM((1, C, 1, W), jnp.float32),
            pltpu.VMEM((1, C, 1, W), jnp.float32),
            pltpu.VMEM((1, 1, 1, W), jnp.float32),
            pltpu.VMEM((2, C, 1, 1), jnp.float32),
        ],
        compiler_params=pltpu.CompilerParams(
            dimension_semantics=("parallel", "arbitrary"),
            vmem_limit_bytes=52 * 1024 * 1024,
        ),
        name="masked_in_fused",
    )(x, w4, b4)
    return out


# speedup vs baseline: 1.0360x; 1.0360x over previous
import jax
import jax.numpy as jnp
from jax.experimental import pallas as pl
from jax.experimental.pallas import tpu as pltpu

_EPS = 1e-05


def _stats_kernel(x_ref, s1_ref, s2_ref, cnt_ref):
    h = pl.program_id(1)
    x = x_ref[...]
    m = (jnp.sum(jnp.abs(x), axis=1, keepdims=True) != 0).astype(x.dtype)
    p1 = jnp.sum(x, axis=2, keepdims=True)
    p2 = jnp.sum(x * x, axis=2, keepdims=True)
    pc = jnp.sum(m, axis=2, keepdims=True)

    @pl.when(h == 0)
    def _():
        s1_ref[...] = p1
        s2_ref[...] = p2
        cnt_ref[...] = pc

    @pl.when(h != 0)
    def _():
        s1_ref[...] += p1
        s2_ref[...] += p2
        cnt_ref[...] += pc


def _apply_kernel(x_ref, s1_ref, s2_ref, cnt_ref, w_ref, b_ref, o_ref):
    x = x_ref[...]
    s1 = jnp.sum(s1_ref[...], axis=3, keepdims=True)
    s2 = jnp.sum(s2_ref[...], axis=3, keepdims=True)
    cnt = jnp.sum(cnt_ref[...], axis=3, keepdims=True)
    w = w_ref[...]
    b = b_ref[...]

    safe = jnp.maximum(cnt, 1.0)
    mean = s1 / safe
    var = jnp.maximum(s2 / safe - mean * mean, 0.0)
    rstd = jax.lax.rsqrt(var + _EPS)
    do_norm = cnt > 1.0
    ws = w * rstd
    scale_v = jnp.where(do_norm, ws, w)
    shift_v = jnp.where(do_norm, b - mean * ws, b)

    m = jnp.sum(jnp.abs(x), axis=1, keepdims=True) != 0
    scale = jnp.where(m, scale_v, w)
    shift = jnp.where(m, shift_v, b)
    o_ref[...] = x * scale + shift


def kernel(x, weight, bias):
    B, C, H, W = x.shape
    hb = min(64, H)
    HB = H // hb
    hbs = min(128, H)
    HBS = H // hbs
    w4 = weight.reshape(1, C, 1, 1)
    b4 = bias.reshape(1, C, 1, 1)

    s1, s2, cnt = pl.pallas_call(
        _stats_kernel,
        grid=(B, HBS),
        in_specs=[pl.BlockSpec((1, C, hbs, W), lambda b, h: (b, 0, h, 0))],
        out_specs=[
            pl.BlockSpec((1, C, 1, W), lambda b, h: (b, 0, 0, 0)),
            pl.BlockSpec((1, C, 1, W), lambda b, h: (b, 0, 0, 0)),
            pl.BlockSpec((1, 1, 1, W), lambda b, h: (b, 0, 0, 0)),
        ],
        out_shape=[
            jax.ShapeDtypeStruct((B, C, 1, W), x.dtype),
            jax.ShapeDtypeStruct((B, C, 1, W), x.dtype),
            jax.ShapeDtypeStruct((B, 1, 1, W), x.dtype),
        ],
        compiler_params=pltpu.CompilerParams(
            dimension_semantics=("parallel", "arbitrary"),
            vmem_limit_bytes=56 * 1024 * 1024,
        ),
        name="masked_in_stats",
    )(x)

    out = pl.pallas_call(
        _apply_kernel,
        grid=(B, HB),
        in_specs=[
            pl.BlockSpec((1, C, hb, W), lambda b, h: (b, 0, h, 0)),
            pl.BlockSpec((1, C, 1, W), lambda b, h: (b, 0, 0, 0)),
            pl.BlockSpec((1, C, 1, W), lambda b, h: (b, 0, 0, 0)),
            pl.BlockSpec((1, 1, 1, W), lambda b, h: (b, 0, 0, 0)),
            pl.BlockSpec((1, C, 1, 1), lambda b, h: (0, 0, 0, 0)),
            pl.BlockSpec((1, C, 1, 1), lambda b, h: (0, 0, 0, 0)),
        ],
        out_specs=pl.BlockSpec((1, C, hb, W), lambda b, h: (b, 0, h, 0)),
        out_shape=jax.ShapeDtypeStruct((B, C, H, W), x.dtype),
        compiler_params=pltpu.CompilerParams(
            dimension_semantics=("parallel", "arbitrary"),
            vmem_limit_bytes=48 * 1024 * 1024,
        ),
        name="masked_in_apply",
    )(x, s1, s2, cnt, w4, b4)
    return out
